# baseline (speedup 1.0000x reference)
import functools

import jax
import jax.numpy as jnp
from jax import lax
from jax.experimental import pallas as pl
from jax.experimental.pallas import tpu as pltpu
from jax.experimental.pallas import tpu_sc as plsc

N = 10000
E = 320000
D = 150
DP = 128
D2 = D - DP
N_PAD = 10240
NT = 16
CH = 128
SB2 = 8
NSB = 10
NCH = 2 * SB2 * NSB
EPT = NCH * CH
E_PAD = NT * EPT
BR = 512
GRID = N_PAD // BR
ROWS_PER_TILE = N_PAD // NT
ZROWS = 8
ZCOPIES = ROWS_PER_TILE // ZROWS

_PREC = None


def _a_body(h_ref, w1t_ref, b1_ref, w2ta_ref, b2a_ref, w2tb_ref, b2b_ref,
            y0_ref, y1_ref):
    t = jnp.dot(h_ref[...], w1t_ref[...], precision=_PREC,
                preferred_element_type=jnp.float32) + b1_ref[...]
    t = jnp.where(t >= 0, t, 0.01 * t)
    y0 = jnp.dot(t, w2ta_ref[...], precision=_PREC,
                 preferred_element_type=jnp.float32) + b2a_ref[...]
    y1 = jnp.dot(t, w2tb_ref[...], precision=_PREC,
                 preferred_element_type=jnp.float32) + b2b_ref[...]
    rows = jax.lax.broadcasted_iota(jnp.int32, y0.shape, 0) + pl.program_id(0) * BR
    y0_ref[...] = jnp.where(rows < N, y0, 0.0)
    y1_ref[...] = jnp.where(rows < N, y1, 0.0)


def _transform(h, w1t, b1, w2ta, b2a, w2tb, b2b):
    blk = pl.BlockSpec((BR, DP), lambda i: (i, 0))
    full = lambda shape: pl.BlockSpec(shape, lambda i: (0, 0))
    return pl.pallas_call(
        _a_body,
        grid=(GRID,),
        in_specs=[pl.BlockSpec((BR, D), lambda i: (i, 0)),
                  full((D, D)), full((1, D)),
                  full((D, DP)), full((1, DP)), full((D, DP)), full((1, DP))],
        out_specs=[blk, blk],
        out_shape=[jax.ShapeDtypeStruct((N_PAD, DP), jnp.float32),
                   jax.ShapeDtypeStruct((N_PAD, DP), jnp.float32)],
    )(h, w1t, b1, w2ta, b2a, w2tb, b2b)


def _sc_body(y0_hbm, y1_hbm, srcs_hbm, dsts_hbm, out_hbm,
             idx_s, idx_d, idx_s2, idx_d2, rows, zbuf, acc, semg0, semg1, semg2):
    cid = lax.axis_index("c")
    sid = lax.axis_index("s")

    def _zrow(i, _):
        def _zcol(j, _):
            zbuf[i, pl.ds(j * 16, 16)] = jnp.zeros((16,), jnp.float32)
            return 0
        return lax.fori_loop(0, DP // 16, _zcol, 0)
    lax.fori_loop(0, ZROWS, _zrow, 0)

    zcps = [
        pltpu.async_copy(
            zbuf, acc.at[pl.ds((sid * ZCOPIES + k) * ZROWS, ZROWS)], semg0)
        for k in range(ZCOPIES)
    ]
    for cp in zcps:
        cp.wait()

    plsc.subcore_barrier()

    def _run(y_hbm):
        def _stage(b, s_buf, d_buf, sem):
            return (pltpu.async_copy(srcs_hbm.at[sid, pl.ds(b * SB2, SB2)],
                                     s_buf, sem),
                    pltpu.async_copy(dsts_hbm.at[sid, pl.ds(b * SB2, SB2)],
                                     d_buf, sem))

        def _sblock(i_s, i_d):
            def _pair(k, _):
                cp0 = pltpu.async_copy(y_hbm.at[i_s.at[k, 0]],
                                       rows.at[0], semg0)
                cp1 = pltpu.async_copy(y_hbm.at[i_s.at[k, 1]],
                                       rows.at[1], semg1)
                cp0.wait()
                cp1.wait()
                pltpu.sync_copy(rows.at[0], acc.at[i_d.at[k, 0]], add=True)
                pltpu.sync_copy(rows.at[1], acc.at[i_d.at[k, 1]], add=True)
                return 0
            lax.fori_loop(0, SB2, _pair, 0)

        bufs = ((idx_s, idx_d), (idx_s2, idx_d2))
        c0, c1 = _stage(0, idx_s, idx_d, semg2)
        c0.wait()
        c1.wait()
        for b in range(NSB):
            if b + 1 < NSB:
                nxt = _stage(b + 1, *bufs[(b + 1) % 2], semg2)
            _sblock(*bufs[b % 2])
            if b + 1 < NSB:
                nxt[0].wait()
                nxt[1].wait()

    @pl.when(cid == 0)
    def _():
        _run(y0_hbm)

    @pl.when(cid == 1)
    def _():
        _run(y1_hbm)

    plsc.subcore_barrier()

    pltpu.sync_copy(acc.at[pl.ds(sid * ROWS_PER_TILE, ROWS_PER_TILE)],
                    out_hbm.at[cid, pl.ds(sid * ROWS_PER_TILE, ROWS_PER_TILE)])


@functools.lru_cache(maxsize=1)
def _make_scatter():
    return pl.kernel(
        _sc_body,
        out_type=jax.ShapeDtypeStruct((2, N_PAD, DP), jnp.float32),
        mesh=plsc.VectorSubcoreMesh(core_axis_name="c", subcore_axis_name="s"),
        scratch_types=[
            pltpu.VMEM((SB2, 2, CH), jnp.int32),
            pltpu.VMEM((SB2, 2, CH), jnp.int32),
            pltpu.VMEM((SB2, 2, CH), jnp.int32),
            pltpu.VMEM((SB2, 2, CH), jnp.int32),
            pltpu.VMEM((2, CH, DP), jnp.float32),
            pltpu.VMEM((ZROWS, DP), jnp.float32),
            pltpu.VMEM_SHARED((N_PAD, DP), jnp.float32),
            pltpu.SemaphoreType.DMA,
            pltpu.SemaphoreType.DMA,
            pltpu.SemaphoreType.DMA,
        ],
    )


def _scatter(y0, y1, srcs, dsts):
    return _make_scatter()(y0, y1, srcs, dsts)


def _ba_body(p0_ref, p1_ref, h_ref, wra_ref, wza_ref, wna_ref,
             wrb_ref, wzb_ref, wnb_ref, ur_ref, uz_ref, un_ref,
             br_ref, bz_ref, bni_ref, bnh_ref,
             w1t_ref, b1_ref, w2ta_ref, b2a_ref, w2tb_ref, b2b_ref,
             hn_ref, y0_ref, y1_ref):
    p0 = p0_ref[...]
    p1 = p1_ref[...]
    h = h_ref[...]

    def dot(x, w):
        return jnp.dot(x, w, precision=_PREC, preferred_element_type=jnp.float32)

    r = jax.nn.sigmoid(dot(p0, wra_ref[...]) + dot(p1, wrb_ref[...])
                       + dot(h, ur_ref[...]) + br_ref[...])
    z = jax.nn.sigmoid(dot(p0, wza_ref[...]) + dot(p1, wzb_ref[...])
                       + dot(h, uz_ref[...]) + bz_ref[...])
    n = jnp.tanh(dot(p0, wna_ref[...]) + dot(p1, wnb_ref[...]) + bni_ref[...]
                 + r * (dot(h, un_ref[...]) + bnh_ref[...]))
    hn = (1.0 - z) * n + z * h
    rows = jax.lax.broadcasted_iota(jnp.int32, hn.shape, 0) + pl.program_id(0) * BR
    hn = jnp.where(rows < N, hn, 0.0)
    hn_ref[...] = hn

    t = dot(hn, w1t_ref[...]) + b1_ref[...]
    t = jnp.where(t >= 0, t, 0.01 * t)
    y0 = dot(t, w2ta_ref[...]) + b2a_ref[...]
    y1 = dot(t, w2tb_ref[...]) + b2b_ref[...]
    rows2 = jax.lax.broadcasted_iota(jnp.int32, y0.shape, 0) + pl.program_id(0) * BR
    y0_ref[...] = jnp.where(rows2 < N, y0, 0.0)
    y1_ref[...] = jnp.where(rows2 < N, y1, 0.0)


def _gru_transform(p0, p1, h, wra, wza, wna, wrb, wzb, wnb, ur, uz, un,
                   br, bz, bni, bnh, w1t, b1, w2ta, b2a, w2tb, b2b):
    full = lambda shape: pl.BlockSpec(shape, lambda i: (0, 0))
    blk = pl.BlockSpec((BR, DP), lambda i: (i, 0))
    blkh = pl.BlockSpec((BR, D), lambda i: (i, 0))
    return pl.pallas_call(
        _ba_body,
        grid=(GRID,),
        in_specs=[blk, blk, blkh,
                  full((DP, D)), full((DP, D)), full((DP, D)),
                  full((DP, D)), full((DP, D)), full((DP, D)),
                  full((D, D)), full((D, D)), full((D, D)),
                  full((1, D)), full((1, D)), full((1, D)), full((1, D)),
                  full((D, D)), full((1, D)),
                  full((D, DP)), full((1, DP)), full((D, DP)), full((1, DP))],
        out_specs=[blkh, blk, blk],
        out_shape=[jax.ShapeDtypeStruct((N_PAD, D), jnp.float32),
                   jax.ShapeDtypeStruct((N_PAD, DP), jnp.float32),
                   jax.ShapeDtypeStruct((N_PAD, DP), jnp.float32)],
    )(p0, p1, h, wra, wza, wna, wrb, wzb, wnb, ur, uz, un,
      br, bz, bni, bnh, w1t, b1, w2ta, b2a, w2tb, b2b)


def _bf_body(p0_ref, p1_ref, h_ref, wra_ref, wza_ref, wna_ref,
             wrb_ref, wzb_ref, wnb_ref, ur_ref, uz_ref, un_ref,
             br_ref, bz_ref, bni_ref, bnh_ref,
             pt_ref, w1g_ref, w1p_ref, fb1_ref, w2_ref, fb2_ref,
             w3_ref, fb3_ref, out_ref, gsum_ref):
    p0 = p0_ref[...]
    p1 = p1_ref[...]
    h = h_ref[...]

    def dot(x, w):
        return jnp.dot(x, w, precision=_PREC, preferred_element_type=jnp.float32)

    r = jax.nn.sigmoid(dot(p0, wra_ref[...]) + dot(p1, wrb_ref[...])
                       + dot(h, ur_ref[...]) + br_ref[...])
    z = jax.nn.sigmoid(dot(p0, wza_ref[...]) + dot(p1, wzb_ref[...])
                       + dot(h, uz_ref[...]) + bz_ref[...])
    n = jnp.tanh(dot(p0, wna_ref[...]) + dot(p1, wnb_ref[...]) + bni_ref[...]
                 + r * (dot(h, un_ref[...]) + bnh_ref[...]))
    hn = (1.0 - z) * n + z * h
    rows = jax.lax.broadcasted_iota(jnp.int32, hn.shape, 0) + pl.program_id(0) * BR
    hn = jnp.where(rows < N, hn, 0.0)

    @pl.when(pl.program_id(0) == 0)
    def _():
        gsum_ref[...] = jnp.zeros_like(gsum_ref)
    gsum_ref[...] += jnp.sum(hn, axis=0, keepdims=True)

    @pl.when(pl.program_id(0) == GRID - 1)
    def _():
        g = gsum_ref[...]
        g = jnp.log(g)
        g = jnp.where(jnp.isnan(g), 0.0, g)
        g = jnp.maximum(g, 0.0)
        x = dot(g, w1g_ref[...]) + pt_ref[...] * w1p_ref[...] + fb1_ref[...]
        x = jnp.where(x >= 0, x, 0.01 * x)
        x = dot(x, w2_ref[...]) + fb2_ref[...]
        x = jnp.where(x >= 0, x, 0.01 * x)
        out_ref[...] = dot(x, w3_ref[...]) + fb3_ref[...]


def _gru_readout(p0, p1, h, wra, wza, wna, wrb, wzb, wnb, ur, uz, un,
                 br, bz, bni, bnh, pt, w1g, w1p, fb1, w2, fb2, w3, fb3):
    full = lambda shape: pl.BlockSpec(shape, lambda i: (0, 0))
    blk = pl.BlockSpec((BR, DP), lambda i: (i, 0))
    blkh = pl.BlockSpec((BR, D), lambda i: (i, 0))
    return pl.pallas_call(
        _bf_body,
        grid=(GRID,),
        in_specs=[blk, blk, blkh,
                  full((DP, D)), full((DP, D)), full((DP, D)),
                  full((DP, D)), full((DP, D)), full((DP, D)),
                  full((D, D)), full((D, D)), full((D, D)),
                  full((1, D)), full((1, D)), full((1, D)), full((1, D)),
                  full((1, 1)), full((D, 80)), full((1, 80)), full((1, 80)),
                  full((80, 80)), full((1, 80)), full((80, 10)),
                  full((1, 10))],
        out_specs=pl.BlockSpec((1, 10), lambda i: (0, 0)),
        out_shape=jax.ShapeDtypeStruct((1, 10), jnp.float32),
        scratch_shapes=[pltpu.VMEM((1, D), jnp.float32)],
    )(p0, p1, h, wra, wza, wna, wrb, wzb, wnb, ur, uz, un,
      br, bz, bni, bnh, pt, w1g, w1p, fb1, w2, fb2, w3, fb3)


def kernel(nodes, edge_index, problem_type, W_e1, b_e1, W_e2, b_e2,
           w_ih, w_hh, b_ih, b_hh, fc1_W, fc1_b, fc2_W, fc2_b, fcl_W, fcl_b):
    f32 = jnp.float32

    w1t = W_e1.T
    b1 = b_e1.reshape(1, D)
    w2t = W_e2.T
    w2ta = w2t[:, :DP]
    b2a = b_e2[:DP].reshape(1, DP)
    w2tb = jnp.zeros((D, DP), f32).at[:, :D2].set(w2t[:, DP:])
    b2b = jnp.zeros((1, DP), f32).at[:, :D2].set(b_e2[DP:])

    w_ihT = w_ih.T
    w_hhT = w_hh.T

    def split_gate(wt, lo):
        a = wt[:DP, lo:lo + D]
        b = jnp.zeros((DP, D), f32).at[:D2].set(wt[DP:, lo:lo + D])
        return a, b

    wra, wrb = split_gate(w_ihT, 0)
    wza, wzb = split_gate(w_ihT, D)
    wna, wnb = split_gate(w_ihT, 2 * D)
    ur = w_hhT[:, 0:D]
    uz = w_hhT[:, D:2 * D]
    un = w_hhT[:, 2 * D:3 * D]
    br = (b_ih[0:D] + b_hh[0:D]).reshape(1, D)
    bz = (b_ih[D:2 * D] + b_hh[D:2 * D]).reshape(1, D)
    bni = b_ih[2 * D:3 * D].reshape(1, D)
    bnh = b_hh[2 * D:3 * D].reshape(1, D)

    pad = jnp.full((E_PAD - E,), N, jnp.int32)
    dsts = jnp.concatenate([edge_index[:, 0], pad]).reshape(NT, NCH // 2, 2, CH)
    srcs = jnp.concatenate([edge_index[:, 1], pad]).reshape(NT, NCH // 2, 2, CH)

    w1g = fc1_W[:, :D].T
    w1p = fc1_W[:, D].reshape(1, -1)
    fb1 = fc1_b.reshape(1, -1)
    w2 = fc2_W.T
    fb2 = fc2_b.reshape(1, -1)
    w3 = fcl_W.T
    fb3 = fcl_b.reshape(1, -1)

    h = jnp.pad(nodes, ((0, N_PAD - N), (0, 0)))
    y0, y1 = _transform(h, w1t, b1, w2ta, b2a, w2tb, b2b)
    for _ in range(2):
        parts = _scatter(y0, y1, srcs, dsts)
        h, y0, y1 = _gru_transform(parts[0], parts[1], h,
                                   wra, wza, wna, wrb, wzb, wnb, ur, uz, un,
                                   br, bz, bni, bnh,
                                   w1t, b1, w2ta, b2a, w2tb, b2b)
    parts = _scatter(y0, y1, srcs, dsts)
    return _gru_readout(parts[0], parts[1], h,
                        wra, wza, wna, wrb, wzb, wnb, ur, uz, un,
                        br, bz, bni, bnh,
                        problem_type, w1g, w1p, fb1, w2, fb2, w3, fb3)


# speedup vs baseline: 1.0086x; 1.0086x over previous
import functools

import jax
import jax.numpy as jnp
from jax import lax
from jax.experimental import pallas as pl
from jax.experimental.pallas import tpu as pltpu
from jax.experimental.pallas import tpu_sc as plsc

N = 10000
E = 320000
D = 150
DP = 128
D2 = D - DP
N_PAD = 10240
NT = 16
CH = 128
SB2 = 16
NSB = 5
NCH = 2 * SB2 * NSB
EPT = NCH * CH
E_PAD = NT * EPT
BR = 512
GRID = N_PAD // BR
ROWS_PER_TILE = N_PAD // NT
ZROWS = 40
ZCOPIES = ROWS_PER_TILE // ZROWS

_PREC = None


def _a_body(h_ref, w1t_ref, b1_ref, w2ta_ref, b2a_ref, w2tb_ref, b2b_ref,
            y0_ref, y1_ref):
    t = jnp.dot(h_ref[...], w1t_ref[...], precision=_PREC,
                preferred_element_type=jnp.float32) + b1_ref[...]
    t = jnp.where(t >= 0, t, 0.01 * t)
    y0 = jnp.dot(t, w2ta_ref[...], precision=_PREC,
                 preferred_element_type=jnp.float32) + b2a_ref[...]
    y1 = jnp.dot(t, w2tb_ref[...], precision=_PREC,
                 preferred_element_type=jnp.float32) + b2b_ref[...]
    rows = jax.lax.broadcasted_iota(jnp.int32, y0.shape, 0) + pl.program_id(0) * BR
    y0_ref[...] = jnp.where(rows < N, y0, 0.0)
    y1_ref[...] = jnp.where(rows < N, y1, 0.0)


def _transform(h, w1t, b1, w2ta, b2a, w2tb, b2b):
    blk = pl.BlockSpec((BR, DP), lambda i: (i, 0))
    full = lambda shape: pl.BlockSpec(shape, lambda i: (0, 0))
    return pl.pallas_call(
        _a_body,
        grid=(GRID,),
        in_specs=[pl.BlockSpec((BR, D), lambda i: (i, 0)),
                  full((D, D)), full((1, D)),
                  full((D, DP)), full((1, DP)), full((D, DP)), full((1, DP))],
        out_specs=[blk, blk],
        out_shape=[jax.ShapeDtypeStruct((N_PAD, DP), jnp.float32),
                   jax.ShapeDtypeStruct((N_PAD, DP), jnp.float32)],
    )(h, w1t, b1, w2ta, b2a, w2tb, b2b)


def _sc_body(y0_hbm, y1_hbm, srcs_hbm, dsts_hbm, out_hbm,
             idx_s, idx_d, rows, zbuf, acc, semg0, semg1):
    cid = lax.axis_index("c")
    sid = lax.axis_index("s")

    def _zrow(i, _):
        def _zcol(j, _):
            zbuf[i, pl.ds(j * 16, 16)] = jnp.zeros((16,), jnp.float32)
            return 0
        return lax.fori_loop(0, DP // 16, _zcol, 0)
    lax.fori_loop(0, ZROWS, _zrow, 0)

    zcps = [
        pltpu.async_copy(
            zbuf, acc.at[pl.ds((sid * ZCOPIES + k) * ZROWS, ZROWS)], semg0)
        for k in range(ZCOPIES)
    ]
    for cp in zcps:
        cp.wait()

    plsc.subcore_barrier()

    def _run(y_hbm):
        def _sblock(b, _):
            ci0 = pltpu.async_copy(
                srcs_hbm.at[sid, pl.ds(b * SB2, SB2)], idx_s, semg0)
            ci1 = pltpu.async_copy(
                dsts_hbm.at[sid, pl.ds(b * SB2, SB2)], idx_d, semg1)
            ci0.wait()
            ci1.wait()

            def _pair(k, _):
                cp0 = pltpu.async_copy(y_hbm.at[idx_s.at[k, 0]],
                                       rows.at[0], semg0)
                cp1 = pltpu.async_copy(y_hbm.at[idx_s.at[k, 1]],
                                       rows.at[1], semg1)
                with jax.named_scope("gwait"):
                    cp0.wait()
                    cp1.wait()
                with jax.named_scope("scat"):
                    pltpu.sync_copy(rows.at[0], acc.at[idx_d.at[k, 0]],
                                    add=True)
                    pltpu.sync_copy(rows.at[1], acc.at[idx_d.at[k, 1]],
                                    add=True)
                return 0
            return lax.fori_loop(0, SB2, _pair, 0)
        lax.fori_loop(0, NSB, _sblock, 0)

    @pl.when(cid == 0)
    def _():
        _run(y0_hbm)

    @pl.when(cid == 1)
    def _():
        _run(y1_hbm)

    plsc.subcore_barrier()

    pltpu.sync_copy(acc.at[pl.ds(sid * ROWS_PER_TILE, ROWS_PER_TILE)],
                    out_hbm.at[cid, pl.ds(sid * ROWS_PER_TILE, ROWS_PER_TILE)])


@functools.lru_cache(maxsize=1)
def _make_scatter():
    return pl.kernel(
        _sc_body,
        out_type=jax.ShapeDtypeStruct((2, N_PAD, DP), jnp.float32),
        mesh=plsc.VectorSubcoreMesh(core_axis_name="c", subcore_axis_name="s"),
        scratch_types=[
            pltpu.VMEM((SB2, 2, CH), jnp.int32),
            pltpu.VMEM((SB2, 2, CH), jnp.int32),
            pltpu.VMEM((2, CH, DP), jnp.float32),
            pltpu.VMEM((ZROWS, DP), jnp.float32),
            pltpu.VMEM_SHARED((N_PAD, DP), jnp.float32),
            pltpu.SemaphoreType.DMA,
            pltpu.SemaphoreType.DMA,
        ],
    )


def _scatter(y0, y1, srcs, dsts):
    return _make_scatter()(y0, y1, srcs, dsts)


def _ba_body(p0_ref, p1_ref, h_ref, wra_ref, wza_ref, wna_ref,
             wrb_ref, wzb_ref, wnb_ref, ur_ref, uz_ref, un_ref,
             br_ref, bz_ref, bni_ref, bnh_ref,
             w1t_ref, b1_ref, w2ta_ref, b2a_ref, w2tb_ref, b2b_ref,
             hn_ref, y0_ref, y1_ref):
    p0 = p0_ref[...]
    p1 = p1_ref[...]
    h = h_ref[...]

    def dot(x, w):
        return jnp.dot(x, w, precision=_PREC, preferred_element_type=jnp.float32)

    r = jax.nn.sigmoid(dot(p0, wra_ref[...]) + dot(p1, wrb_ref[...])
                       + dot(h, ur_ref[...]) + br_ref[...])
    z = jax.nn.sigmoid(dot(p0, wza_ref[...]) + dot(p1, wzb_ref[...])
                       + dot(h, uz_ref[...]) + bz_ref[...])
    n = jnp.tanh(dot(p0, wna_ref[...]) + dot(p1, wnb_ref[...]) + bni_ref[...]
                 + r * (dot(h, un_ref[...]) + bnh_ref[...]))
    hn = (1.0 - z) * n + z * h
    rows = jax.lax.broadcasted_iota(jnp.int32, hn.shape, 0) + pl.program_id(0) * BR
    hn = jnp.where(rows < N, hn, 0.0)
    hn_ref[...] = hn

    t = dot(hn, w1t_ref[...]) + b1_ref[...]
    t = jnp.where(t >= 0, t, 0.01 * t)
    y0 = dot(t, w2ta_ref[...]) + b2a_ref[...]
    y1 = dot(t, w2tb_ref[...]) + b2b_ref[...]
    rows2 = jax.lax.broadcasted_iota(jnp.int32, y0.shape, 0) + pl.program_id(0) * BR
    y0_ref[...] = jnp.where(rows2 < N, y0, 0.0)
    y1_ref[...] = jnp.where(rows2 < N, y1, 0.0)


def _gru_transform(p0, p1, h, wra, wza, wna, wrb, wzb, wnb, ur, uz, un,
                   br, bz, bni, bnh, w1t, b1, w2ta, b2a, w2tb, b2b):
    full = lambda shape: pl.BlockSpec(shape, lambda i: (0, 0))
    blk = pl.BlockSpec((BR, DP), lambda i: (i, 0))
    blkh = pl.BlockSpec((BR, D), lambda i: (i, 0))
    return pl.pallas_call(
        _ba_body,
        grid=(GRID,),
        in_specs=[blk, blk, blkh,
                  full((DP, D)), full((DP, D)), full((DP, D)),
                  full((DP, D)), full((DP, D)), full((DP, D)),
                  full((D, D)), full((D, D)), full((D, D)),
                  full((1, D)), full((1, D)), full((1, D)), full((1, D)),
                  full((D, D)), full((1, D)),
                  full((D, DP)), full((1, DP)), full((D, DP)), full((1, DP))],
        out_specs=[blkh, blk, blk],
        out_shape=[jax.ShapeDtypeStruct((N_PAD, D), jnp.float32),
                   jax.ShapeDtypeStruct((N_PAD, DP), jnp.float32),
                   jax.ShapeDtypeStruct((N_PAD, DP), jnp.float32)],
    )(p0, p1, h, wra, wza, wna, wrb, wzb, wnb, ur, uz, un,
      br, bz, bni, bnh, w1t, b1, w2ta, b2a, w2tb, b2b)


def _bf_body(p0_ref, p1_ref, h_ref, wra_ref, wza_ref, wna_ref,
             wrb_ref, wzb_ref, wnb_ref, ur_ref, uz_ref, un_ref,
             br_ref, bz_ref, bni_ref, bnh_ref,
             pt_ref, w1g_ref, w1p_ref, fb1_ref, w2_ref, fb2_ref,
             w3_ref, fb3_ref, out_ref, gsum_ref):
    p0 = p0_ref[...]
    p1 = p1_ref[...]
    h = h_ref[...]

    def dot(x, w):
        return jnp.dot(x, w, precision=_PREC, preferred_element_type=jnp.float32)

    r = jax.nn.sigmoid(dot(p0, wra_ref[...]) + dot(p1, wrb_ref[...])
                       + dot(h, ur_ref[...]) + br_ref[...])
    z = jax.nn.sigmoid(dot(p0, wza_ref[...]) + dot(p1, wzb_ref[...])
                       + dot(h, uz_ref[...]) + bz_ref[...])
    n = jnp.tanh(dot(p0, wna_ref[...]) + dot(p1, wnb_ref[...]) + bni_ref[...]
                 + r * (dot(h, un_ref[...]) + bnh_ref[...]))
    hn = (1.0 - z) * n + z * h
    rows = jax.lax.broadcasted_iota(jnp.int32, hn.shape, 0) + pl.program_id(0) * BR
    hn = jnp.where(rows < N, hn, 0.0)

    @pl.when(pl.program_id(0) == 0)
    def _():
        gsum_ref[...] = jnp.zeros_like(gsum_ref)
    gsum_ref[...] += jnp.sum(hn, axis=0, keepdims=True)

    @pl.when(pl.program_id(0) == GRID - 1)
    def _():
        g = gsum_ref[...]
        g = jnp.log(g)
        g = jnp.where(jnp.isnan(g), 0.0, g)
        g = jnp.maximum(g, 0.0)
        x = dot(g, w1g_ref[...]) + pt_ref[...] * w1p_ref[...] + fb1_ref[...]
        x = jnp.where(x >= 0, x, 0.01 * x)
        x = dot(x, w2_ref[...]) + fb2_ref[...]
        x = jnp.where(x >= 0, x, 0.01 * x)
        out_ref[...] = dot(x, w3_ref[...]) + fb3_ref[...]


def _gru_readout(p0, p1, h, wra, wza, wna, wrb, wzb, wnb, ur, uz, un,
                 br, bz, bni, bnh, pt, w1g, w1p, fb1, w2, fb2, w3, fb3):
    full = lambda shape: pl.BlockSpec(shape, lambda i: (0, 0))
    blk = pl.BlockSpec((BR, DP), lambda i: (i, 0))
    blkh = pl.BlockSpec((BR, D), lambda i: (i, 0))
    return pl.pallas_call(
        _bf_body,
        grid=(GRID,),
        in_specs=[blk, blk, blkh,
                  full((DP, D)), full((DP, D)), full((DP, D)),
                  full((DP, D)), full((DP, D)), full((DP, D)),
                  full((D, D)), full((D, D)), full((D, D)),
                  full((1, D)), full((1, D)), full((1, D)), full((1, D)),
                  full((1, 1)), full((D, 80)), full((1, 80)), full((1, 80)),
                  full((80, 80)), full((1, 80)), full((80, 10)),
                  full((1, 10))],
        out_specs=pl.BlockSpec((1, 10), lambda i: (0, 0)),
        out_shape=jax.ShapeDtypeStruct((1, 10), jnp.float32),
        scratch_shapes=[pltpu.VMEM((1, D), jnp.float32)],
    )(p0, p1, h, wra, wza, wna, wrb, wzb, wnb, ur, uz, un,
      br, bz, bni, bnh, pt, w1g, w1p, fb1, w2, fb2, w3, fb3)


def kernel(nodes, edge_index, problem_type, W_e1, b_e1, W_e2, b_e2,
           w_ih, w_hh, b_ih, b_hh, fc1_W, fc1_b, fc2_W, fc2_b, fcl_W, fcl_b):
    f32 = jnp.float32

    w1t = W_e1.T
    b1 = b_e1.reshape(1, D)
    w2t = W_e2.T
    w2ta = w2t[:, :DP]
    b2a = b_e2[:DP].reshape(1, DP)
    w2tb = jnp.zeros((D, DP), f32).at[:, :D2].set(w2t[:, DP:])
    b2b = jnp.zeros((1, DP), f32).at[:, :D2].set(b_e2[DP:])

    w_ihT = w_ih.T
    w_hhT = w_hh.T

    def split_gate(wt, lo):
        a = wt[:DP, lo:lo + D]
        b = jnp.zeros((DP, D), f32).at[:D2].set(wt[DP:, lo:lo + D])
        return a, b

    wra, wrb = split_gate(w_ihT, 0)
    wza, wzb = split_gate(w_ihT, D)
    wna, wnb = split_gate(w_ihT, 2 * D)
    ur = w_hhT[:, 0:D]
    uz = w_hhT[:, D:2 * D]
    un = w_hhT[:, 2 * D:3 * D]
    br = (b_ih[0:D] + b_hh[0:D]).reshape(1, D)
    bz = (b_ih[D:2 * D] + b_hh[D:2 * D]).reshape(1, D)
    bni = b_ih[2 * D:3 * D].reshape(1, D)
    bnh = b_hh[2 * D:3 * D].reshape(1, D)

    pad = jnp.full((E_PAD - E,), N, jnp.int32)
    dsts = jnp.concatenate([edge_index[:, 0], pad]).reshape(NT, NCH // 2, 2, CH)
    srcs = jnp.concatenate([edge_index[:, 1], pad]).reshape(NT, NCH // 2, 2, CH)

    w1g = fc1_W[:, :D].T
    w1p = fc1_W[:, D].reshape(1, -1)
    fb1 = fc1_b.reshape(1, -1)
    w2 = fc2_W.T
    fb2 = fc2_b.reshape(1, -1)
    w3 = fcl_W.T
    fb3 = fcl_b.reshape(1, -1)

    h = jnp.pad(nodes, ((0, N_PAD - N), (0, 0)))
    y0, y1 = _transform(h, w1t, b1, w2ta, b2a, w2tb, b2b)
    for _ in range(2):
        parts = _scatter(y0, y1, srcs, dsts)
        h, y0, y1 = _gru_transform(parts[0], parts[1], h,
                                   wra, wza, wna, wrb, wzb, wnb, ur, uz, un,
                                   br, bz, bni, bnh,
                                   w1t, b1, w2ta, b2a, w2tb, b2b)
    parts = _scatter(y0, y1, srcs, dsts)
    return _gru_readout(parts[0], parts[1], h,
                        wra, wza, wna, wrb, wzb, wnb, ur, uz, un,
                        br, bz, bni, bnh,
                        problem_type, w1g, w1p, fb1, w2, fb2, w3, fb3)
